# baseline (speedup 1.0000x reference)
import functools

import jax
import jax.numpy as jnp
from jax import lax
from jax.experimental import pallas as pl
from jax.experimental.pallas import tpu as pltpu
from jax.experimental.pallas import tpu_sc as plsc

BATCH = 4
SEQ = 4096
D_MODEL = 2048
SEQ_BLK = 512


def _sc_gather(control_table, code_ids):
    mesh = plsc.VectorSubcoreMesh(core_axis_name="c", subcore_axis_name="s")

    @functools.partial(
        pl.kernel,
        mesh=mesh,
        out_type=jax.ShapeDtypeStruct((BATCH, D_MODEL), jnp.float32),
        scratch_types=[
            pltpu.VMEM((BATCH,), jnp.int32),
            pltpu.VMEM((BATCH, D_MODEL), jnp.float32),
            pltpu.SemaphoreType.DMA,
        ],
    )
    def gather_kernel(table_hbm, idx_hbm, out_hbm, idx_v, rows_v, sem):
        wid = lax.axis_index("s") * 2 + lax.axis_index("c")

        @pl.when(wid == 0)
        def _():
            pltpu.sync_copy(idx_hbm, idx_v)
            pltpu.async_copy(table_hbm.at[idx_v], rows_v, sem).wait()
            pltpu.sync_copy(rows_v, out_hbm)

    return gather_kernel(control_table, code_ids.astype(jnp.int32))


def _add_body(x_ref, row_ref, o_ref):
    o_ref[...] = x_ref[...] + row_ref[...]


def kernel(x, code_ids, control_table):
    rows = _sc_gather(control_table, code_ids)[:, None, :]
    grid = (BATCH, SEQ // SEQ_BLK)
    return pl.pallas_call(
        _add_body,
        grid=grid,
        in_specs=[
            pl.BlockSpec((1, SEQ_BLK, D_MODEL), lambda b, s: (b, s, 0)),
            pl.BlockSpec((1, 1, D_MODEL), lambda b, s: (b, 0, 0)),
        ],
        out_specs=pl.BlockSpec((1, SEQ_BLK, D_MODEL), lambda b, s: (b, s, 0)),
        out_shape=jax.ShapeDtypeStruct(x.shape, x.dtype),
    )(x, rows)


# speedup vs baseline: 1.0194x; 1.0194x over previous
import functools

import jax
import jax.numpy as jnp
from jax import lax
from jax.experimental import pallas as pl
from jax.experimental.pallas import tpu as pltpu
from jax.experimental.pallas import tpu_sc as plsc

BATCH = 4
SEQ = 4096
D_MODEL = 2048
SEQ_BLK = 1024


def _sc_gather(control_table, code_ids):
    mesh = plsc.VectorSubcoreMesh(core_axis_name="c", subcore_axis_name="s")

    @functools.partial(
        pl.kernel,
        mesh=mesh,
        out_type=jax.ShapeDtypeStruct((BATCH, D_MODEL), jnp.float32),
        scratch_types=[
            pltpu.VMEM((BATCH,), jnp.int32),
            pltpu.VMEM((BATCH, D_MODEL), jnp.float32),
            pltpu.SemaphoreType.DMA,
        ],
    )
    def gather_kernel(table_hbm, idx_hbm, out_hbm, idx_v, rows_v, sem):
        wid = lax.axis_index("s") * 2 + lax.axis_index("c")

        @pl.when(wid == 0)
        def _():
            pltpu.sync_copy(idx_hbm, idx_v)
            pltpu.async_copy(table_hbm.at[idx_v], rows_v, sem).wait()
            pltpu.sync_copy(rows_v, out_hbm)

    return gather_kernel(control_table, code_ids.astype(jnp.int32))


def _add_body(x_ref, row_ref, o_ref):
    o_ref[...] = x_ref[...] + row_ref[...]


def kernel(x, code_ids, control_table):
    rows = _sc_gather(control_table, code_ids)[:, None, :]
    grid = (BATCH, SEQ // SEQ_BLK)
    return pl.pallas_call(
        _add_body,
        grid=grid,
        in_specs=[
            pl.BlockSpec((1, SEQ_BLK, D_MODEL), lambda b, s: (b, s, 0)),
            pl.BlockSpec((1, 1, D_MODEL), lambda b, s: (b, 0, 0)),
        ],
        out_specs=pl.BlockSpec((1, SEQ_BLK, D_MODEL), lambda b, s: (b, s, 0)),
        out_shape=jax.ShapeDtypeStruct(x.shape, x.dtype),
    )(x, rows)


# speedup vs baseline: 1.1696x; 1.1473x over previous
import jax
import jax.numpy as jnp
from jax.experimental import pallas as pl
from jax.experimental.pallas import tpu as pltpu

BATCH = 4
SEQ = 4096
D_MODEL = 2048
SEQ_BLK = 1024


def _add_body(ids_ref, x_ref, row_ref, o_ref):
    o_ref[...] = x_ref[...] + row_ref[...]


def kernel(x, code_ids, control_table):
    grid = (BATCH, SEQ // SEQ_BLK)
    grid_spec = pltpu.PrefetchScalarGridSpec(
        num_scalar_prefetch=1,
        grid=grid,
        in_specs=[
            pl.BlockSpec((1, SEQ_BLK, D_MODEL), lambda b, s, ids: (b, s, 0)),
            pl.BlockSpec((1, 1, D_MODEL), lambda b, s, ids: (ids[b], 0, 0)),
        ],
        out_specs=pl.BlockSpec((1, SEQ_BLK, D_MODEL), lambda b, s, ids: (b, s, 0)),
    )
    return pl.pallas_call(
        _add_body,
        grid_spec=grid_spec,
        out_shape=jax.ShapeDtypeStruct(x.shape, x.dtype),
    )(code_ids.astype(jnp.int32), x, control_table[:, None, :])


# speedup vs baseline: 1.2932x; 1.1057x over previous
import jax
import jax.numpy as jnp
from jax import lax
from jax.experimental import pallas as pl
from jax.experimental.pallas import tpu as pltpu

BATCH = 4
SEQ = 4096
D_MODEL = 2048
ROWS = BATCH * SEQ
CHUNK = 512
NBUF = 4
NCHUNKS = ROWS // CHUNK
NOUTER = NCHUNKS // NBUF
CH_PER_B = SEQ // CHUNK


def _body(ids_ref, x_hbm, tab_hbm, o_hbm, rows_v, in_buf, out_buf,
          row_sem, in_sems, out_sems):
    def in_copy(i, k):
        return pltpu.make_async_copy(
            x_hbm.at[pl.ds(i * CHUNK, CHUNK), :], in_buf.at[k], in_sems.at[k])

    def out_copy(i, k):
        return pltpu.make_async_copy(
            out_buf.at[k], o_hbm.at[pl.ds(i * CHUNK, CHUNK), :], out_sems.at[k])

    for b in range(BATCH):
        pltpu.make_async_copy(
            tab_hbm.at[pl.ds(ids_ref[b], 1), :], rows_v.at[pl.ds(b, 1), :],
            row_sem).start()
    for k in range(NBUF):
        in_copy(k, k).start()
    for b in range(BATCH):
        pltpu.make_async_copy(
            tab_hbm.at[pl.ds(ids_ref[b], 1), :], rows_v.at[pl.ds(b, 1), :],
            row_sem).wait()

    def outer_step(outer, _):
        for k in range(NBUF):
            i = outer * NBUF + k
            in_copy(i, k).wait()

            @pl.when(outer > 0)
            def _():
                out_copy(i - NBUF, k).wait()

            b = i // CH_PER_B
            out_buf[k] = in_buf[k] + rows_v[pl.ds(b, 1), :]
            out_copy(i, k).start()

            @pl.when(outer < NOUTER - 1)
            def _():
                in_copy(i + NBUF, k).start()
        return 0

    lax.fori_loop(0, NOUTER, outer_step, 0)
    for k in range(NBUF):
        out_copy(NCHUNKS - NBUF + k, k).wait()


def kernel(x, code_ids, control_table):
    grid_spec = pltpu.PrefetchScalarGridSpec(
        num_scalar_prefetch=1,
        grid=(1,),
        in_specs=[
            pl.BlockSpec(memory_space=pl.ANY),
            pl.BlockSpec(memory_space=pl.ANY),
        ],
        out_specs=pl.BlockSpec(memory_space=pl.ANY),
        scratch_shapes=[
            pltpu.VMEM((BATCH, D_MODEL), jnp.float32),
            pltpu.VMEM((NBUF, CHUNK, D_MODEL), jnp.float32),
            pltpu.VMEM((NBUF, CHUNK, D_MODEL), jnp.float32),
            pltpu.SemaphoreType.DMA,
            pltpu.SemaphoreType.DMA((NBUF,)),
            pltpu.SemaphoreType.DMA((NBUF,)),
        ],
    )
    out = pl.pallas_call(
        _body,
        grid_spec=grid_spec,
        out_shape=jax.ShapeDtypeStruct((ROWS, D_MODEL), x.dtype),
    )(code_ids.astype(jnp.int32), x.reshape(ROWS, D_MODEL), control_table)
    return out.reshape(x.shape)
